# baseline (speedup 1.0000x reference)
import functools

import jax
import jax.numpy as jnp
from jax import lax
from jax.experimental import pallas as pl
from jax.experimental.pallas import tpu as pltpu
from jax.experimental.pallas import tpu_sc as plsc

_NS = 16
_L = 16


def _tec_body(B, C, inputs_hbm, targets_hbm, out_hbm, stage_hbm,
              tgt_v, win_v, part_v, red_v, sem):
    rpt = B // _NS
    wid = lax.axis_index("s")
    base = wid * rpt

    pltpu.sync_copy(targets_hbm.at[pl.ds(base, rpt)], tgt_v)

    copies = []
    for g in range(rpt // _L):
        tv = tgt_v[pl.ds(g * _L, _L)]
        colv = lax.shift_left(lax.shift_right_logical(tv, 7), 7)
        for l in range(_L):
            r = g * _L + l
            colb = pl.multiple_of(colv[l], 128)
            rowb = pl.multiple_of(base + (r & ~7), 8)
            cp = pltpu.make_async_copy(
                inputs_hbm.at[pl.ds(rowb, 8), pl.ds(colb, 128)],
                win_v.at[r], sem)
            cp.start()
            copies.append(cp)
    for cp in copies:
        cp.wait()

    lanes = lax.iota(jnp.int32, _L)
    acc = jnp.zeros((_L,), jnp.float32)
    for g in range(rpt // _L):
        tv = tgt_v[pl.ds(g * _L, _L)]
        acc = acc + plsc.load_gather(
            win_v,
            [lanes + jnp.int32(g * _L), lanes & jnp.int32(7),
             tv & jnp.int32(127)])

    part_v[...] = acc
    pltpu.sync_copy(part_v, stage_hbm.at[wid])
    plsc.subcore_barrier()

    @pl.when(wid == 0)
    def _():
        pltpu.sync_copy(stage_hbm, red_v)
        tot = jnp.zeros((_L,), jnp.float32)
        for w in range(_NS):
            tot = tot + red_v[w]
        s = jnp.sum(tot)
        part_v[...] = (jnp.zeros((_L,), jnp.float32) + s) * jnp.float32(-1.0 / B)
        pltpu.sync_copy(part_v, out_hbm)


@jax.jit
def kernel(inputs, targets):
    B, C = inputs.shape
    rpt = B // _NS
    mesh = plsc.VectorSubcoreMesh(
        core_axis_name="c", subcore_axis_name="s", num_cores=1)
    out, _ = pl.kernel(
        functools.partial(_tec_body, B, C),
        out_type=(jax.ShapeDtypeStruct((_L,), jnp.float32),
                  jax.ShapeDtypeStruct((_NS, _L), jnp.float32)),
        mesh=mesh,
        compiler_params=pltpu.CompilerParams(needs_layout_passes=False),
        scratch_types=[
            pltpu.VMEM((rpt,), jnp.int32),
            pltpu.VMEM((rpt, 8, 128), jnp.float32),
            pltpu.VMEM((_L,), jnp.float32),
            pltpu.VMEM((_NS, _L), jnp.float32),
            pltpu.SemaphoreType.DMA,
        ],
    )(inputs, targets.astype(jnp.int32))
    return out[0]
